# baseline (speedup 1.0000x reference)
import functools

import jax
import jax.numpy as jnp
from jax import lax
from jax.experimental import pallas as pl
from jax.experimental.pallas import tpu as pltpu
from jax.experimental.pallas import tpu_sc as plsc

NC = 2
NS = 16
NW = NC * NS
PASSES = 2
CHUNK = 128


def _make_sc_scores(V, D, B, NCTX):
    S = B // (NW * PASSES)
    KC = S // CHUNK
    NWV = NW * PASSES
    mesh = plsc.VectorSubcoreMesh(core_axis_name="c", subcore_axis_name="s")

    def prep_idx(idx, off):
        for k in range(KC):
            for l in range(CHUNK // 16):
                sl = pl.ds(l * 16, 16)
                v = idx[k, sl]
                idx[k, sl] = lax.shift_right_logical(v, 1)
                off[pl.ds(k * CHUNK + l * 16, 16)] = (
                    lax.shift_left(jnp.bitwise_and(v, 1), 6)
                )

    def fire(emb, idx, rows, sem):
        for k in range(KC):
            pltpu.async_copy(emb.at[idx.at[k]], rows.at[pl.ds(k * CHUNK, CHUNK)], sem)

    def drain(emb, idx, rows, sem):
        for k in range(KC):
            pltpu.make_async_copy(
                emb.at[idx.at[k]], rows.at[pl.ds(k * CHUNK, CHUNK)], sem
            ).wait()

    @functools.partial(
        pl.kernel,
        out_type=jax.ShapeDtypeStruct((NCTX * B,), jnp.float32),
        mesh=mesh,
        compiler_params=pltpu.CompilerParams(
            needs_layout_passes=False, use_tc_tiling_on_sc=True
        ),
        scratch_types=[
            pltpu.VMEM((KC, CHUNK), jnp.int32),
            pltpu.VMEM((KC, CHUNK), jnp.int32),
            pltpu.VMEM((KC, CHUNK), jnp.int32),
            pltpu.VMEM((S,), jnp.int32),
            pltpu.VMEM((S,), jnp.int32),
            pltpu.VMEM((S,), jnp.int32),
            pltpu.VMEM((S, 2 * D), jnp.float32),
            pltpu.VMEM((S, 2 * D), jnp.float32),
            pltpu.VMEM((S, 2 * D), jnp.float32),
            pltpu.VMEM((3 * CHUNK,), jnp.float32),
            pltpu.VMEM((S,), jnp.float32),
            pltpu.SemaphoreType.DMA,
            pltpu.SemaphoreType.DMA,
            pltpu.SemaphoreType.DMA,
        ],
    )
    def sc_scores(cw_hbm, ctx_hbm, in_emb, out_emb, out_hbm,
                  cidx, xidx0, xidx1, coff, xoff0, xoff1,
                  crow, xrow0, xrow1, tmp, srow, csem, sem0, sem1):
        wid = lax.axis_index("s") * NC + lax.axis_index("c")
        rid17 = lax.iota(jnp.int32, 16) * 17

        for p in range(PASSES):
            vw = wid * PASSES + p
            wbase = vw * S

            def compute_chunk(xrow, xoff, j):
                @pl.loop(0, S // 16)
                def _(g):
                    b0 = g * 16
                    cov = coff[pl.ds(b0, 16)]
                    xov = xoff[pl.ds(b0, 16)]
                    for e in range(16):
                        b = b0 + e
                        co = cov[e]
                        xo = xov[e]
                        v = crow[b, pl.ds(co, 16)] * xrow[b, pl.ds(xo, 16)]
                        for q in range(1, D // 16):
                            v = v + (crow[b, pl.ds(co + q * 16, 16)]
                                     * xrow[b, pl.ds(xo + q * 16, 16)])
                        tmp[pl.ds(e * 17, 16)] = v
                    acc = plsc.load_gather(tmp, [rid17])
                    for c in range(1, 16):
                        acc = acc + plsc.load_gather(tmp, [rid17 + c])
                    srow[pl.ds(b0, 16)] = acc
                off = pl.multiple_of(j * B + wbase, S)
                pltpu.sync_copy(srow, out_hbm.at[pl.ds(off, S)])

            pltpu.sync_copy(cw_hbm.at[vw], cidx)
            prep_idx(cidx, coff)
            fire(in_emb, cidx, crow, csem)
            pltpu.sync_copy(ctx_hbm.at[0, vw], xidx0)
            prep_idx(xidx0, xoff0)
            fire(out_emb, xidx0, xrow0, sem0)
            drain(in_emb, cidx, crow, csem)

            @pl.loop(0, NCTX - 1, step=2)
            def _(j):
                pltpu.sync_copy(ctx_hbm.at[j + 1, vw], xidx1)
                prep_idx(xidx1, xoff1)
                fire(out_emb, xidx1, xrow1, sem1)
                drain(out_emb, xidx0, xrow0, sem0)
                compute_chunk(xrow0, xoff0, j)
                pltpu.sync_copy(ctx_hbm.at[j + 2, vw], xidx0)
                prep_idx(xidx0, xoff0)
                fire(out_emb, xidx0, xrow0, sem0)
                drain(out_emb, xidx1, xrow1, sem1)
                compute_chunk(xrow1, xoff1, j + 1)

            drain(out_emb, xidx0, xrow0, sem0)
            compute_chunk(xrow0, xoff0, NCTX - 1)

    return sc_scores


def _make_tc_loss(B, NEG):
    def body(s_ref, o_ref):
        s = s_ref[...]
        row = lax.broadcasted_iota(jnp.int32, s.shape, 0)
        x = jnp.where(row == 0, s, -s)
        ls = jax.nn.log_sigmoid(x)
        w = jnp.where(row == 0, 1.0 / B, 1.0 / (B * NEG))
        o_ref[0, 0] = -jnp.sum(ls * w)

    return pl.pallas_call(
        body,
        out_shape=jax.ShapeDtypeStruct((1, 1), jnp.float32),
        out_specs=pl.BlockSpec(memory_space=pltpu.SMEM),
    )


def kernel(center_words, positive_context, negative_context, input_emb, output_emb):
    B = center_words.shape[0]
    NEG = negative_context.shape[1]
    V, D = input_emb.shape
    NCTX = NEG + 1
    NWV = NW * PASSES
    S = B // NWV

    cw = center_words.astype(jnp.int32).reshape(NWV, S // CHUNK, CHUNK)
    ctx = jnp.concatenate(
        [positive_context[None, :], negative_context.T], axis=0
    ).astype(jnp.int32).reshape(NCTX, NWV, S // CHUNK, CHUNK)
    in128 = input_emb.reshape(V // 2, 2 * D)
    out128 = output_emb.reshape(V // 2, 2 * D)

    scores = _make_sc_scores(V, D, B, NCTX)(cw, ctx, in128, out128)
    loss = _make_tc_loss(B, NEG)(scores.reshape(NCTX, B))
    return loss[0, 0]
